# baseline (speedup 1.0000x reference)
import functools

import jax
import jax.numpy as jnp
from jax import lax
from jax.experimental import pallas as pl
from jax.experimental.pallas import tpu as pltpu
from jax.experimental.pallas import tpu_sc as plsc

DAMPING = 0.1
EPS = 1e-5
ODE_STEPS = 4
LANES = 128
RB = 12
CH = 4
SUB = RB * LANES // CH
HALF = 16


def _weights_body(wpos_ref, wneg_ref, wpsi_ref, bpos_ref, bneg_ref,
                  ucat_ref, cvec_ref):
    h = wpos_ref.shape[0]
    psi1 = wpsi_ref[:h, :]
    psi2 = wpsi_ref[h:, :]
    upos = jnp.dot(wpos_ref[...], psi1, preferred_element_type=jnp.float32)
    uneg = jnp.dot(wneg_ref[...], psi2, preferred_element_type=jnp.float32)
    ucat_ref[...] = jnp.concatenate([upos, uneg], axis=1)
    cvec_ref[...] = (
        jnp.dot(bpos_ref[...], psi1, preferred_element_type=jnp.float32)
        + jnp.dot(bneg_ref[...], psi2, preferred_element_type=jnp.float32))


def _fold_weights(W_pos, W_neg, W_psi, b_pos, b_neg):
    h = W_pos.shape[0]
    return pl.pallas_call(
        _weights_body,
        out_shape=[jax.ShapeDtypeStruct((h, 2 * h), jnp.float32),
                   jax.ShapeDtypeStruct((1, h), jnp.float32)],
    )(W_pos, W_neg, W_psi, b_pos.reshape(1, h), b_neg.reshape(1, h))


def _encoder_body(x_ref, w_ref, b_ref, g_ref, bb_ref, out_ref):
    z = jnp.dot(x_ref[...], w_ref[...], preferred_element_type=jnp.float32)
    z = z + b_ref[...]
    mu = jnp.mean(z, axis=1, keepdims=True)
    var = jnp.mean((z - mu) * (z - mu), axis=1, keepdims=True)
    out_ref[...] = (z - mu) * lax.rsqrt(var + EPS) * g_ref[...] + bb_ref[...]


def _encode(x, W_enc, b_enc, fn_g, fn_b, bn):
    n, f = x.shape
    h = W_enc.shape[1]
    return pl.pallas_call(
        _encoder_body,
        grid=(-(-n // bn),),
        in_specs=[
            pl.BlockSpec((bn, f), lambda i: (i, 0)),
            pl.BlockSpec((f, h), lambda i: (0, 0)),
            pl.BlockSpec((1, h), lambda i: (0, 0)),
            pl.BlockSpec((1, h), lambda i: (0, 0)),
            pl.BlockSpec((1, h), lambda i: (0, 0)),
        ],
        out_specs=pl.BlockSpec((bn, h), lambda i: (i, 0)),
        out_shape=jax.ShapeDtypeStruct((n, h), jnp.float32),
    )(x, W_enc, b_enc.reshape(1, h), fn_g.reshape(1, h), fn_b.reshape(1, h))


def _pre_body(e0_ref, e1_ref, degp_ref, degn_ref, amat_ref, bd_ref,
              g0_ref, g1_ref, b0_ref, b1_ref,
              tp0_ref, tp1_ref, tn0_ref, tn1_ref):
    e0 = e0_ref[...]
    e1 = e1_ref[...]
    amat = amat_ref[...]
    dot = lambda a, b: jnp.dot(a, b, preferred_element_type=jnp.float32)
    mu = dot(e0, amat) + dot(e1, amat)
    xc0 = e0 - mu
    xc1 = e1 - mu
    var = dot(xc0 * xc0, amat) + dot(xc1 * xc1, amat)
    r = lax.rsqrt(var + EPS)
    hn0 = xc0 * r * g0_ref[...] + b0_ref[...]
    hn1 = xc1 * r * g1_ref[...] + b1_ref[...]
    disp = lax.rsqrt(degp_ref[...])
    disn = lax.rsqrt(degn_ref[...])
    tp0_ref[...] = (dot(hn0, bd_ref[0]) + dot(hn1, bd_ref[1])) * disp
    tp1_ref[...] = (dot(hn0, bd_ref[2]) + dot(hn1, bd_ref[3])) * disp
    tn0_ref[...] = (dot(hn0, bd_ref[4]) + dot(hn1, bd_ref[5])) * disn
    tn1_ref[...] = (dot(hn0, bd_ref[6]) + dot(hn1, bd_ref[7])) * disn


def _pre_stage(ev0, ev1, degp8, degn8, amat, bd, g0, g1, b0, b1, bm):
    m = ev0.shape[0]
    grid = (m // bm,)
    v = pl.BlockSpec((bm, LANES), lambda i: (i, 0))
    w1 = pl.BlockSpec((1, LANES), lambda i: (0, 0))
    sds = jax.ShapeDtypeStruct((m, LANES), jnp.float32)
    return pl.pallas_call(
        _pre_body,
        grid=grid,
        in_specs=[v, v, v, v,
                  pl.BlockSpec((LANES, LANES), lambda i: (0, 0)),
                  pl.BlockSpec((8, LANES, LANES), lambda i: (0, 0, 0)),
                  w1, w1, w1, w1],
        out_specs=[v, v, v, v],
        out_shape=[sds, sds, sds, sds],
    )(ev0, ev1, degp8, degn8, amat, bd, g0, g1, b0, b1)


def _post_body(sp0_ref, sp1_ref, sn0_ref, sn1_ref, degp_ref, degn_ref,
               e0_ref, e1_ref, a0_ref, a1_ref, z0_ref, z1_ref,
               c0_ref, c1_ref, coef_ref,
               ao0_ref, ao1_ref, nx0_ref, nx1_ref):
    disp = lax.rsqrt(degp_ref[...])
    disn = lax.rsqrt(degn_ref[...])
    w = coef_ref[0, 0]
    a = coef_ref[0, 1]
    f0 = jnp.clip(
        jnp.tanh(disp * sp0_ref[...] + disn * sn0_ref[...] + c0_ref[...])
        - DAMPING * e0_ref[...], -50.0, 50.0)
    f1 = jnp.clip(
        jnp.tanh(disp * sp1_ref[...] + disn * sn1_ref[...] + c1_ref[...])
        - DAMPING * e1_ref[...], -50.0, 50.0)
    ao0_ref[...] = a0_ref[...] + w * f0
    ao1_ref[...] = a1_ref[...] + w * f1
    nx0_ref[...] = z0_ref[...] + a * f0
    nx1_ref[...] = z1_ref[...] + a * f1


def _post_stage(sp0, sp1, sn0, sn1, degp8, degn8, ev0, ev1, acc0, acc1,
                base0, base1, c0, c1, coef, bm):
    m = sp0.shape[0]
    grid = (m // bm,)
    v = pl.BlockSpec((bm, LANES), lambda i: (i, 0))
    w1 = pl.BlockSpec((1, LANES), lambda i: (0, 0))
    sds = jax.ShapeDtypeStruct((m, LANES), jnp.float32)
    return pl.pallas_call(
        _post_body,
        grid=grid,
        in_specs=[v, v, v, v, v, v, v, v, v, v, v, v, w1, w1,
                  pl.BlockSpec((1, 2), lambda i: (0, 0))],
        out_specs=[v, v, v, v],
        out_shape=[sds, sds, sds, sds],
    )(sp0, sp1, sn0, sn1, degp8, degn8, ev0, ev1, acc0, acc1,
      base0, base1, c0, c1, coef)


@functools.lru_cache(maxsize=None)
def _make_deg_kernel(npad, nblk, ns):
    iters = -(-nblk // ns)
    rpt = npad // ns
    mesh = plsc.VectorSubcoreMesh(core_axis_name="c", subcore_axis_name="s")
    out_sds = jax.ShapeDtypeStruct((npad, HALF), jnp.float32)

    @functools.partial(
        pl.kernel,
        out_type=[out_sds, out_sds],
        mesh=mesh,
        scratch_types=[
            pltpu.VMEM((RB * LANES,), jnp.int32),
            pltpu.VMEM((RB * LANES, HALF), jnp.float32),
            pltpu.VMEM_SHARED((npad, HALF), jnp.float32),
            pltpu.SemaphoreType.DMA,
        ],
        compiler_params=pltpu.CompilerParams(use_tc_tiling_on_sc=False),
    )
    def deg_kernel(dst_hbm, ones_hbm, outp_hbm, outn_hbm,
                   idxd_v, ones_v, acc, sem):
        c = lax.axis_index("c")
        sid = lax.axis_index("s")

        def run(dst2, out):
            pltpu.sync_copy(ones_hbm.at[pl.ds(0, RB * LANES)], ones_v)
            pltpu.sync_copy(ones_hbm.at[pl.ds(sid * rpt, rpt)],
                            acc.at[pl.ds(sid * rpt, rpt)])
            plsc.subcore_barrier()

            def block_body(i, _):
                b = sid + i * ns

                @pl.when(b < nblk)
                def _():
                    pltpu.sync_copy(dst2.at[pl.ds(b * RB * LANES,
                                                  RB * LANES)], idxd_v)
                    pltpu.async_copy(ones_v, acc.at[idxd_v], sem,
                                     add=True).wait()
                return 0

            lax.fori_loop(0, iters, block_body, 0)
            plsc.subcore_barrier()
            pltpu.sync_copy(acc.at[pl.ds(sid * rpt, rpt)],
                            out.at[pl.ds(sid * rpt, rpt)])

        @pl.when(c == 0)
        def _():
            run(dst_hbm.at[0], outp_hbm)

        @pl.when(c == 1)
        def _():
            run(dst_hbm.at[1], outn_hbm)

    return deg_kernel


@functools.lru_cache(maxsize=None)
def _make_msg_kernel(npad, nblk, ns):
    iters = -(-nblk // ns)
    rpt = npad // ns
    mesh = plsc.VectorSubcoreMesh(core_axis_name="c", subcore_axis_name="s")
    out_sds = jax.ShapeDtypeStruct((npad, HALF), jnp.float32)

    @functools.partial(
        pl.kernel,
        out_type=[out_sds, out_sds, out_sds, out_sds],
        mesh=mesh,
        scratch_types=[
            [pltpu.VMEM((SUB,), jnp.int32)] * CH,
            [pltpu.VMEM((SUB,), jnp.int32)] * CH,
            pltpu.VMEM((RB * LANES, HALF), jnp.float32),
            pltpu.VMEM_SHARED((npad, HALF), jnp.float32),
            pltpu.SemaphoreType.DMA,
            pltpu.SemaphoreType.DMA,
        ],
        compiler_params=pltpu.CompilerParams(use_tc_tiling_on_sc=False),
    )
    def msg_kernel(tp0_hbm, tp1_hbm, tn0_hbm, tn1_hbm, src_hbm, dst_hbm,
                   sp0_hbm, sp1_hbm, sn0_hbm, sn1_hbm,
                   idxs_v, idxd_v, rows_v, acc, sem_g, sem_s):
        c = lax.axis_index("c")
        sid = lax.axis_index("s")

        def run_sign(tbl, out, s):
            pltpu.sync_copy(tbl.at[pl.ds(sid * rpt, rpt)],
                            acc.at[pl.ds(sid * rpt, rpt)])
            plsc.subcore_barrier()

            def block_body(i, _):
                b = sid + i * ns

                @pl.when(b < nblk)
                def _():
                    eo = b * RB * LANES
                    gd = []
                    for k in range(CH):
                        pltpu.sync_copy(
                            src_hbm.at[s].at[pl.ds(eo + k * SUB, SUB)],
                            idxs_v[k])
                        gd.append(pltpu.async_copy(
                            tbl.at[idxs_v[k]],
                            rows_v.at[pl.ds(k * SUB, SUB)], sem_g))
                    for k in range(CH):
                        pltpu.sync_copy(
                            dst_hbm.at[s].at[pl.ds(eo + k * SUB, SUB)],
                            idxd_v[k])
                    sd = []
                    for k in range(CH):
                        gd[k].wait()
                        sd.append(pltpu.async_copy(
                            rows_v.at[pl.ds(k * SUB, SUB)],
                            acc.at[idxd_v[k]], sem_s, add=True))
                    for d in sd:
                        d.wait()
                return 0

            lax.fori_loop(0, iters, block_body, 0)
            plsc.subcore_barrier()
            pltpu.sync_copy(acc.at[pl.ds(sid * rpt, rpt)],
                            out.at[pl.ds(sid * rpt, rpt)])
            plsc.subcore_barrier()

        @pl.when(c == 0)
        def _():
            run_sign(tp0_hbm, sp0_hbm, 0)
            run_sign(tn0_hbm, sn0_hbm, 1)

        @pl.when(c == 1)
        def _():
            run_sign(tp1_hbm, sp1_hbm, 0)
            run_sign(tn1_hbm, sn1_hbm, 1)

    return msg_kernel


def _tile8(vec16):
    return jnp.tile(vec16, 8).reshape(1, LANES)


def kernel(x, edge_index_pos, edge_index_neg, t, W_enc, b_enc, fn_g, fn_b,
           ln_g, ln_b, W_pos, b_pos, W_neg, b_neg, W_psi):
    n, _ = x.shape
    h = W_enc.shape[1]
    e = edge_index_pos.shape[1]
    blk_e = RB * LANES
    nblk = -(-e // blk_e)
    epad = nblk * blk_e - e
    npad = -(-n // LANES) * LANES
    m = npad * HALF // LANES
    info = plsc.get_sparse_core_info()
    ns = info.num_subcores
    bm = min(m, 1088)
    while m % bm or bm % 8:
        bm -= 1

    src_all = jnp.stack([edge_index_pos[0], edge_index_neg[0]])
    dst_all = jnp.stack([edge_index_pos[1], edge_index_neg[1]])
    if epad:
        src_all = jnp.pad(src_all, ((0, 0), (0, epad)))
        dst_all = jnp.pad(dst_all, ((0, 0), (0, epad)),
                          constant_values=jnp.int32(n))
    src_all = src_all.reshape(2, nblk * RB * LANES)
    dst_all = dst_all.reshape(2, nblk * RB * LANES)
    ones_p = jnp.ones((m, LANES), jnp.float32).reshape(npad, HALF)

    degp, degn = _make_deg_kernel(npad, nblk, ns)(dst_all, ones_p)
    degp8 = degp.reshape(m, LANES)
    degn8 = degn.reshape(m, LANES)

    ucat, cvec = _fold_weights(W_pos, W_neg, W_psi, b_pos, b_neg)
    upos, uneg = ucat[:, :h], ucat[:, h:]
    eye8 = jnp.eye(8, dtype=jnp.float32)
    amat = jnp.kron(eye8, jnp.full((HALF, HALF), 1.0 / h, jnp.float32))
    bd = jnp.stack([
        jnp.kron(eye8, upos[:HALF, :HALF]),
        jnp.kron(eye8, upos[HALF:, :HALF]),
        jnp.kron(eye8, upos[:HALF, HALF:]),
        jnp.kron(eye8, upos[HALF:, HALF:]),
        jnp.kron(eye8, uneg[:HALF, :HALF]),
        jnp.kron(eye8, uneg[HALF:, :HALF]),
        jnp.kron(eye8, uneg[:HALF, HALF:]),
        jnp.kron(eye8, uneg[HALF:, HALF:]),
    ])
    g0 = _tile8(ln_g[:HALF])
    g1 = _tile8(ln_g[HALF:])
    b0 = _tile8(ln_b[:HALF])
    b1 = _tile8(ln_b[HALF:])
    c0 = _tile8(cvec[0, :HALF])
    c1 = _tile8(cvec[0, HALF:])

    h0 = _encode(x, W_enc, b_enc, fn_g, fn_b, 2000)
    pad_rows = ((0, npad - n), (0, 0))
    ev0 = jnp.pad(h0[:, :HALF], pad_rows).reshape(m, LANES)
    ev1 = jnp.pad(h0[:, HALF:], pad_rows).reshape(m, LANES)

    msg = _make_msg_kernel(npad, nblk, ns)

    dt = (t[1] - t[0]) / ODE_STEPS
    wts = (dt / 6.0, dt / 3.0, dt / 3.0, dt / 6.0)
    ats = (dt / 2.0, dt / 2.0, dt, dt * 0.0)

    for _ in range(ODE_STEPS):
        base0, base1 = ev0, ev1
        acc0, acc1 = base0, base1
        for si in range(4):
            tp0, tp1, tn0, tn1 = _pre_stage(
                ev0, ev1, degp8, degn8, amat, bd, g0, g1, b0, b1, bm)
            sp0, sp1, sn0, sn1 = msg(
                tp0.reshape(npad, HALF), tp1.reshape(npad, HALF),
                tn0.reshape(npad, HALF), tn1.reshape(npad, HALF),
                src_all, dst_all)
            coef = jnp.stack([wts[si], ats[si]]).reshape(1, 2)
            acc0, acc1, ev0, ev1 = _post_stage(
                sp0.reshape(m, LANES), sp1.reshape(m, LANES),
                sn0.reshape(m, LANES), sn1.reshape(m, LANES),
                degp8, degn8, ev0, ev1, acc0, acc1, base0, base1,
                c0, c1, coef, bm)
        ev0, ev1 = acc0, acc1

    hv0 = ev0.reshape(npad, HALF)[:n]
    hv1 = ev1.reshape(npad, HALF)[:n]
    return jnp.concatenate([hv0, hv1], axis=1)


# speedup vs baseline: 1.2294x; 1.2294x over previous
import functools

import jax
import jax.numpy as jnp
from jax import lax
from jax.experimental import pallas as pl
from jax.experimental.pallas import tpu as pltpu
from jax.experimental.pallas import tpu_sc as plsc

DAMPING = 0.1
EPS = 1e-5
ODE_STEPS = 4
LANES = 128
RB = 12
CH = 4
SUB = RB * LANES // CH
HALF = 16


def _weights_body(wpos_ref, wneg_ref, wpsi_ref, bpos_ref, bneg_ref,
                  ucat_ref, cvec_ref):
    h = wpos_ref.shape[0]
    psi1 = wpsi_ref[:h, :]
    psi2 = wpsi_ref[h:, :]
    upos = jnp.dot(wpos_ref[...], psi1, preferred_element_type=jnp.float32)
    uneg = jnp.dot(wneg_ref[...], psi2, preferred_element_type=jnp.float32)
    ucat_ref[...] = jnp.concatenate([upos, uneg], axis=1)
    cvec_ref[...] = (
        jnp.dot(bpos_ref[...], psi1, preferred_element_type=jnp.float32)
        + jnp.dot(bneg_ref[...], psi2, preferred_element_type=jnp.float32))


def _fold_weights(W_pos, W_neg, W_psi, b_pos, b_neg):
    h = W_pos.shape[0]
    return pl.pallas_call(
        _weights_body,
        out_shape=[jax.ShapeDtypeStruct((h, 2 * h), jnp.float32),
                   jax.ShapeDtypeStruct((1, h), jnp.float32)],
    )(W_pos, W_neg, W_psi, b_pos.reshape(1, h), b_neg.reshape(1, h))


def _encoder_body(x_ref, w_ref, b_ref, g_ref, bb_ref, out_ref):
    z = jnp.dot(x_ref[...], w_ref[...], preferred_element_type=jnp.float32)
    z = z + b_ref[...]
    mu = jnp.mean(z, axis=1, keepdims=True)
    var = jnp.mean((z - mu) * (z - mu), axis=1, keepdims=True)
    out_ref[...] = (z - mu) * lax.rsqrt(var + EPS) * g_ref[...] + bb_ref[...]


def _encode(x, W_enc, b_enc, fn_g, fn_b, bn):
    n, f = x.shape
    h = W_enc.shape[1]
    return pl.pallas_call(
        _encoder_body,
        grid=(-(-n // bn),),
        in_specs=[
            pl.BlockSpec((bn, f), lambda i: (i, 0)),
            pl.BlockSpec((f, h), lambda i: (0, 0)),
            pl.BlockSpec((1, h), lambda i: (0, 0)),
            pl.BlockSpec((1, h), lambda i: (0, 0)),
            pl.BlockSpec((1, h), lambda i: (0, 0)),
        ],
        out_specs=pl.BlockSpec((bn, h), lambda i: (i, 0)),
        out_shape=jax.ShapeDtypeStruct((n, h), jnp.float32),
    )(x, W_enc, b_enc.reshape(1, h), fn_g.reshape(1, h), fn_b.reshape(1, h))


def _pre_body(e0_ref, e1_ref, degp_ref, degn_ref, amat_ref, bd_ref,
              g0_ref, g1_ref, b0_ref, b1_ref,
              tp0_ref, tp1_ref, tn0_ref, tn1_ref):
    e0 = e0_ref[...]
    e1 = e1_ref[...]
    amat = amat_ref[...]
    dot = lambda a, b: jnp.dot(a, b, preferred_element_type=jnp.float32)
    mu = dot(e0, amat) + dot(e1, amat)
    xc0 = e0 - mu
    xc1 = e1 - mu
    var = dot(xc0 * xc0, amat) + dot(xc1 * xc1, amat)
    r = lax.rsqrt(var + EPS)
    hn0 = xc0 * r * g0_ref[...] + b0_ref[...]
    hn1 = xc1 * r * g1_ref[...] + b1_ref[...]
    disp = lax.rsqrt(degp_ref[...])
    disn = lax.rsqrt(degn_ref[...])
    tp0_ref[...] = (dot(hn0, bd_ref[0]) + dot(hn1, bd_ref[1])) * disp
    tp1_ref[...] = (dot(hn0, bd_ref[2]) + dot(hn1, bd_ref[3])) * disp
    tn0_ref[...] = (dot(hn0, bd_ref[4]) + dot(hn1, bd_ref[5])) * disn
    tn1_ref[...] = (dot(hn0, bd_ref[6]) + dot(hn1, bd_ref[7])) * disn


def _pre_stage(ev0, ev1, degp8, degn8, amat, bd, g0, g1, b0, b1, bm):
    m = ev0.shape[0]
    grid = (m // bm,)
    v = pl.BlockSpec((bm, LANES), lambda i: (i, 0))
    w1 = pl.BlockSpec((1, LANES), lambda i: (0, 0))
    sds = jax.ShapeDtypeStruct((m, LANES), jnp.float32)
    return pl.pallas_call(
        _pre_body,
        grid=grid,
        in_specs=[v, v, v, v,
                  pl.BlockSpec((LANES, LANES), lambda i: (0, 0)),
                  pl.BlockSpec((8, LANES, LANES), lambda i: (0, 0, 0)),
                  w1, w1, w1, w1],
        out_specs=[v, v, v, v],
        out_shape=[sds, sds, sds, sds],
    )(ev0, ev1, degp8, degn8, amat, bd, g0, g1, b0, b1)


def _post_body(sp0_ref, sp1_ref, sn0_ref, sn1_ref, degp_ref, degn_ref,
               e0_ref, e1_ref, a0_ref, a1_ref, z0_ref, z1_ref,
               c0_ref, c1_ref, coef_ref,
               ao0_ref, ao1_ref, nx0_ref, nx1_ref):
    disp = lax.rsqrt(degp_ref[...])
    disn = lax.rsqrt(degn_ref[...])
    w = coef_ref[0, 0]
    a = coef_ref[0, 1]
    f0 = jnp.clip(
        jnp.tanh(disp * sp0_ref[...] + disn * sn0_ref[...] + c0_ref[...])
        - DAMPING * e0_ref[...], -50.0, 50.0)
    f1 = jnp.clip(
        jnp.tanh(disp * sp1_ref[...] + disn * sn1_ref[...] + c1_ref[...])
        - DAMPING * e1_ref[...], -50.0, 50.0)
    ao0_ref[...] = a0_ref[...] + w * f0
    ao1_ref[...] = a1_ref[...] + w * f1
    nx0_ref[...] = z0_ref[...] + a * f0
    nx1_ref[...] = z1_ref[...] + a * f1


def _post_stage(sp0, sp1, sn0, sn1, degp8, degn8, ev0, ev1, acc0, acc1,
                base0, base1, c0, c1, coef, bm):
    m = sp0.shape[0]
    grid = (m // bm,)
    v = pl.BlockSpec((bm, LANES), lambda i: (i, 0))
    w1 = pl.BlockSpec((1, LANES), lambda i: (0, 0))
    sds = jax.ShapeDtypeStruct((m, LANES), jnp.float32)
    return pl.pallas_call(
        _post_body,
        grid=grid,
        in_specs=[v, v, v, v, v, v, v, v, v, v, v, v, w1, w1,
                  pl.BlockSpec((1, 2), lambda i: (0, 0))],
        out_specs=[v, v, v, v],
        out_shape=[sds, sds, sds, sds],
    )(sp0, sp1, sn0, sn1, degp8, degn8, ev0, ev1, acc0, acc1,
      base0, base1, c0, c1, coef)


@functools.lru_cache(maxsize=None)
def _make_deg_kernel(npad, nblk, ns):
    iters = -(-nblk // ns)
    rpt = npad // ns
    mesh = plsc.VectorSubcoreMesh(core_axis_name="c", subcore_axis_name="s")
    out_sds = jax.ShapeDtypeStruct((npad, HALF), jnp.float32)

    @functools.partial(
        pl.kernel,
        out_type=[out_sds, out_sds],
        mesh=mesh,
        scratch_types=[
            pltpu.VMEM((RB * LANES,), jnp.int32),
            pltpu.VMEM((RB * LANES, HALF), jnp.float32),
            pltpu.VMEM_SHARED((npad, HALF), jnp.float32),
            pltpu.SemaphoreType.DMA,
        ],
        compiler_params=pltpu.CompilerParams(use_tc_tiling_on_sc=False),
    )
    def deg_kernel(dst_hbm, ones_hbm, outp_hbm, outn_hbm,
                   idxd_v, ones_v, acc, sem):
        c = lax.axis_index("c")
        sid = lax.axis_index("s")

        def run(dst2, out):
            pltpu.sync_copy(ones_hbm.at[pl.ds(0, RB * LANES)], ones_v)
            pltpu.sync_copy(ones_hbm.at[pl.ds(sid * rpt, rpt)],
                            acc.at[pl.ds(sid * rpt, rpt)])
            plsc.subcore_barrier()

            def block_body(i, _):
                b = sid + i * ns

                @pl.when(b < nblk)
                def _():
                    pltpu.sync_copy(dst2.at[pl.ds(b * RB * LANES,
                                                  RB * LANES)], idxd_v)
                    pltpu.async_copy(ones_v, acc.at[idxd_v], sem,
                                     add=True).wait()
                return 0

            lax.fori_loop(0, iters, block_body, 0)
            plsc.subcore_barrier()
            pltpu.sync_copy(acc.at[pl.ds(sid * rpt, rpt)],
                            out.at[pl.ds(sid * rpt, rpt)])

        @pl.when(c == 0)
        def _():
            run(dst_hbm.at[0], outp_hbm)

        @pl.when(c == 1)
        def _():
            run(dst_hbm.at[1], outn_hbm)

    return deg_kernel


@functools.lru_cache(maxsize=None)
def _make_msg_kernel(npad, nblk, ns):
    iters = -(-nblk // ns)
    rpt = npad // ns
    mesh = plsc.VectorSubcoreMesh(core_axis_name="c", subcore_axis_name="s")
    out_sds = jax.ShapeDtypeStruct((npad, HALF), jnp.float32)

    @functools.partial(
        pl.kernel,
        out_type=[out_sds, out_sds, out_sds, out_sds],
        mesh=mesh,
        scratch_types=[
            pltpu.VMEM((RB, LANES), jnp.int32),
            pltpu.VMEM((RB, LANES), jnp.int32),
            pltpu.VMEM((RB * LANES, HALF), jnp.float32),
            pltpu.VMEM_SHARED((npad, HALF), jnp.float32),
            pltpu.SemaphoreType.DMA,
            pltpu.SemaphoreType.DMA,
        ],
        compiler_params=pltpu.CompilerParams(use_tc_tiling_on_sc=False),
    )
    def msg_kernel(tp0_hbm, tp1_hbm, tn0_hbm, tn1_hbm, src_hbm, dst_hbm,
                   sp0_hbm, sp1_hbm, sn0_hbm, sn1_hbm,
                   idxs_v, idxd_v, rows_v, acc, sem_g, sem_s):
        c = lax.axis_index("c")
        sid = lax.axis_index("s")

        def run_sign(tbl, out, s):
            pltpu.sync_copy(tbl.at[pl.ds(sid * rpt, rpt)],
                            acc.at[pl.ds(sid * rpt, rpt)])
            plsc.subcore_barrier()

            def block_body(i, _):
                b = sid + i * ns

                @pl.when(b < nblk)
                def _():
                    pltpu.sync_copy(src_hbm.at[s].at[b], idxs_v)
                    pltpu.sync_copy(dst_hbm.at[s].at[b], idxd_v)
                    gd = [
                        pltpu.async_copy(
                            tbl.at[idxs_v.at[j]],
                            rows_v.at[pl.ds(j * LANES, LANES)], sem_g)
                        for j in range(RB)
                    ]
                    sd = []
                    for j in range(RB):
                        gd[j].wait()
                        sd.append(pltpu.async_copy(
                            rows_v.at[pl.ds(j * LANES, LANES)],
                            acc.at[idxd_v.at[j]], sem_s, add=True))
                    for d in sd:
                        d.wait()
                return 0

            lax.fori_loop(0, iters, block_body, 0)
            plsc.subcore_barrier()
            pltpu.sync_copy(acc.at[pl.ds(sid * rpt, rpt)],
                            out.at[pl.ds(sid * rpt, rpt)])
            plsc.subcore_barrier()

        @pl.when(c == 0)
        def _():
            run_sign(tp0_hbm, sp0_hbm, 0)
            run_sign(tn0_hbm, sn0_hbm, 1)

        @pl.when(c == 1)
        def _():
            run_sign(tp1_hbm, sp1_hbm, 0)
            run_sign(tn1_hbm, sn1_hbm, 1)

    return msg_kernel


def _tile8(vec16):
    return jnp.tile(vec16, 8).reshape(1, LANES)


def kernel(x, edge_index_pos, edge_index_neg, t, W_enc, b_enc, fn_g, fn_b,
           ln_g, ln_b, W_pos, b_pos, W_neg, b_neg, W_psi):
    n, _ = x.shape
    h = W_enc.shape[1]
    e = edge_index_pos.shape[1]
    blk_e = RB * LANES
    nblk = -(-e // blk_e)
    epad = nblk * blk_e - e
    npad = -(-n // LANES) * LANES
    m = npad * HALF // LANES
    info = plsc.get_sparse_core_info()
    ns = info.num_subcores
    bm = min(m, 1088)
    while m % bm or bm % 8:
        bm -= 1

    src_all = jnp.stack([edge_index_pos[0], edge_index_neg[0]])
    dst_all = jnp.stack([edge_index_pos[1], edge_index_neg[1]])
    if epad:
        src_all = jnp.pad(src_all, ((0, 0), (0, epad)))
        dst_all = jnp.pad(dst_all, ((0, 0), (0, epad)),
                          constant_values=jnp.int32(n))
    src_4d = src_all.reshape(2, nblk, RB, LANES)
    dst_4d = dst_all.reshape(2, nblk, RB, LANES)
    dst_flat = dst_all.reshape(2, nblk * RB * LANES)
    ones_p = jnp.ones((m, LANES), jnp.float32).reshape(npad, HALF)

    degp, degn = _make_deg_kernel(npad, nblk, ns)(dst_flat, ones_p)
    degp8 = degp.reshape(m, LANES)
    degn8 = degn.reshape(m, LANES)

    ucat, cvec = _fold_weights(W_pos, W_neg, W_psi, b_pos, b_neg)
    upos, uneg = ucat[:, :h], ucat[:, h:]
    eye8 = jnp.eye(8, dtype=jnp.float32)
    amat = jnp.kron(eye8, jnp.full((HALF, HALF), 1.0 / h, jnp.float32))
    bd = jnp.stack([
        jnp.kron(eye8, upos[:HALF, :HALF]),
        jnp.kron(eye8, upos[HALF:, :HALF]),
        jnp.kron(eye8, upos[:HALF, HALF:]),
        jnp.kron(eye8, upos[HALF:, HALF:]),
        jnp.kron(eye8, uneg[:HALF, :HALF]),
        jnp.kron(eye8, uneg[HALF:, :HALF]),
        jnp.kron(eye8, uneg[:HALF, HALF:]),
        jnp.kron(eye8, uneg[HALF:, HALF:]),
    ])
    g0 = _tile8(ln_g[:HALF])
    g1 = _tile8(ln_g[HALF:])
    b0 = _tile8(ln_b[:HALF])
    b1 = _tile8(ln_b[HALF:])
    c0 = _tile8(cvec[0, :HALF])
    c1 = _tile8(cvec[0, HALF:])

    h0 = _encode(x, W_enc, b_enc, fn_g, fn_b, 2000)
    pad_rows = ((0, npad - n), (0, 0))
    ev0 = jnp.pad(h0[:, :HALF], pad_rows).reshape(m, LANES)
    ev1 = jnp.pad(h0[:, HALF:], pad_rows).reshape(m, LANES)

    msg = _make_msg_kernel(npad, nblk, ns)

    dt = (t[1] - t[0]) / ODE_STEPS
    wts = (dt / 6.0, dt / 3.0, dt / 3.0, dt / 6.0)
    ats = (dt / 2.0, dt / 2.0, dt, dt * 0.0)

    for _ in range(ODE_STEPS):
        base0, base1 = ev0, ev1
        acc0, acc1 = base0, base1
        for si in range(4):
            tp0, tp1, tn0, tn1 = _pre_stage(
                ev0, ev1, degp8, degn8, amat, bd, g0, g1, b0, b1, bm)
            sp0, sp1, sn0, sn1 = msg(
                tp0.reshape(npad, HALF), tp1.reshape(npad, HALF),
                tn0.reshape(npad, HALF), tn1.reshape(npad, HALF),
                src_4d, dst_4d)
            coef = jnp.stack([wts[si], ats[si]]).reshape(1, 2)
            acc0, acc1, ev0, ev1 = _post_stage(
                sp0.reshape(m, LANES), sp1.reshape(m, LANES),
                sn0.reshape(m, LANES), sn1.reshape(m, LANES),
                degp8, degn8, ev0, ev1, acc0, acc1, base0, base1,
                c0, c1, coef, bm)
        ev0, ev1 = acc0, acc1

    hv0 = ev0.reshape(npad, HALF)[:n]
    hv1 = ev1.reshape(npad, HALF)[:n]
    return jnp.concatenate([hv0, hv1], axis=1)


# speedup vs baseline: 1.2306x; 1.0009x over previous
import functools

import jax
import jax.numpy as jnp
from jax import lax
from jax.experimental import pallas as pl
from jax.experimental.pallas import tpu as pltpu
from jax.experimental.pallas import tpu_sc as plsc

DAMPING = 0.1
EPS = 1e-5
ODE_STEPS = 4
LANES = 128
RB = 12
HALF = 16


def _weights_body(wpos_ref, wneg_ref, wpsi_ref, bpos_ref, bneg_ref,
                  ucat_ref, cvec_ref):
    h = wpos_ref.shape[0]
    psi1 = wpsi_ref[:h, :]
    psi2 = wpsi_ref[h:, :]
    upos = jnp.dot(wpos_ref[...], psi1, preferred_element_type=jnp.float32)
    uneg = jnp.dot(wneg_ref[...], psi2, preferred_element_type=jnp.float32)
    ucat_ref[...] = jnp.concatenate([upos, uneg], axis=1)
    cvec_ref[...] = (
        jnp.dot(bpos_ref[...], psi1, preferred_element_type=jnp.float32)
        + jnp.dot(bneg_ref[...], psi2, preferred_element_type=jnp.float32))


def _fold_weights(W_pos, W_neg, W_psi, b_pos, b_neg):
    h = W_pos.shape[0]
    return pl.pallas_call(
        _weights_body,
        out_shape=[jax.ShapeDtypeStruct((h, 2 * h), jnp.float32),
                   jax.ShapeDtypeStruct((1, h), jnp.float32)],
    )(W_pos, W_neg, W_psi, b_pos.reshape(1, h), b_neg.reshape(1, h))


def _encoder_body(x_ref, w_ref, b_ref, g_ref, bb_ref, out_ref):
    z = jnp.dot(x_ref[...], w_ref[...], preferred_element_type=jnp.float32)
    z = z + b_ref[...]
    mu = jnp.mean(z, axis=1, keepdims=True)
    var = jnp.mean((z - mu) * (z - mu), axis=1, keepdims=True)
    out_ref[...] = (z - mu) * lax.rsqrt(var + EPS) * g_ref[...] + bb_ref[...]


def _encode(x, W_enc, b_enc, fn_g, fn_b, bn):
    n, f = x.shape
    h = W_enc.shape[1]
    return pl.pallas_call(
        _encoder_body,
        grid=(-(-n // bn),),
        in_specs=[
            pl.BlockSpec((bn, f), lambda i: (i, 0)),
            pl.BlockSpec((f, h), lambda i: (0, 0)),
            pl.BlockSpec((1, h), lambda i: (0, 0)),
            pl.BlockSpec((1, h), lambda i: (0, 0)),
            pl.BlockSpec((1, h), lambda i: (0, 0)),
        ],
        out_specs=pl.BlockSpec((bn, h), lambda i: (i, 0)),
        out_shape=jax.ShapeDtypeStruct((n, h), jnp.float32),
    )(x, W_enc, b_enc.reshape(1, h), fn_g.reshape(1, h), fn_b.reshape(1, h))


def _pre_body(e0_ref, e1_ref, degp_ref, degn_ref, amat_ref, bd_ref,
              g0_ref, g1_ref, b0_ref, b1_ref,
              tp0_ref, tp1_ref, tn0_ref, tn1_ref):
    e0 = e0_ref[...]
    e1 = e1_ref[...]
    amat = amat_ref[...]
    dot = lambda a, b: jnp.dot(a, b, preferred_element_type=jnp.float32)
    mu = dot(e0, amat) + dot(e1, amat)
    xc0 = e0 - mu
    xc1 = e1 - mu
    var = dot(xc0 * xc0, amat) + dot(xc1 * xc1, amat)
    r = lax.rsqrt(var + EPS)
    hn0 = xc0 * r * g0_ref[...] + b0_ref[...]
    hn1 = xc1 * r * g1_ref[...] + b1_ref[...]
    disp = lax.rsqrt(degp_ref[...])
    disn = lax.rsqrt(degn_ref[...])
    tp0_ref[...] = (dot(hn0, bd_ref[0]) + dot(hn1, bd_ref[1])) * disp
    tp1_ref[...] = (dot(hn0, bd_ref[2]) + dot(hn1, bd_ref[3])) * disp
    tn0_ref[...] = (dot(hn0, bd_ref[4]) + dot(hn1, bd_ref[5])) * disn
    tn1_ref[...] = (dot(hn0, bd_ref[6]) + dot(hn1, bd_ref[7])) * disn


def _pre_stage(ev0, ev1, degp8, degn8, amat, bd, g0, g1, b0, b1, bm):
    m = ev0.shape[0]
    grid = (m // bm,)
    v = pl.BlockSpec((bm, LANES), lambda i: (i, 0))
    w1 = pl.BlockSpec((1, LANES), lambda i: (0, 0))
    sds = jax.ShapeDtypeStruct((m, LANES), jnp.float32)
    return pl.pallas_call(
        _pre_body,
        grid=grid,
        in_specs=[v, v, v, v,
                  pl.BlockSpec((LANES, LANES), lambda i: (0, 0)),
                  pl.BlockSpec((8, LANES, LANES), lambda i: (0, 0, 0)),
                  w1, w1, w1, w1],
        out_specs=[v, v, v, v],
        out_shape=[sds, sds, sds, sds],
    )(ev0, ev1, degp8, degn8, amat, bd, g0, g1, b0, b1)


def _post_body(sp0_ref, sp1_ref, sn0_ref, sn1_ref, degp_ref, degn_ref,
               e0_ref, e1_ref, a0_ref, a1_ref, z0_ref, z1_ref,
               c0_ref, c1_ref, coef_ref,
               ao0_ref, ao1_ref, nx0_ref, nx1_ref):
    disp = lax.rsqrt(degp_ref[...])
    disn = lax.rsqrt(degn_ref[...])
    w = coef_ref[0, 0]
    a = coef_ref[0, 1]
    f0 = jnp.clip(
        jnp.tanh(disp * sp0_ref[...] + disn * sn0_ref[...] + c0_ref[...])
        - DAMPING * e0_ref[...], -50.0, 50.0)
    f1 = jnp.clip(
        jnp.tanh(disp * sp1_ref[...] + disn * sn1_ref[...] + c1_ref[...])
        - DAMPING * e1_ref[...], -50.0, 50.0)
    ao0_ref[...] = a0_ref[...] + w * f0
    ao1_ref[...] = a1_ref[...] + w * f1
    nx0_ref[...] = z0_ref[...] + a * f0
    nx1_ref[...] = z1_ref[...] + a * f1


def _post_stage(sp0, sp1, sn0, sn1, degp8, degn8, ev0, ev1, acc0, acc1,
                base0, base1, c0, c1, coef, bm):
    m = sp0.shape[0]
    grid = (m // bm,)
    v = pl.BlockSpec((bm, LANES), lambda i: (i, 0))
    w1 = pl.BlockSpec((1, LANES), lambda i: (0, 0))
    sds = jax.ShapeDtypeStruct((m, LANES), jnp.float32)
    return pl.pallas_call(
        _post_body,
        grid=grid,
        in_specs=[v, v, v, v, v, v, v, v, v, v, v, v, w1, w1,
                  pl.BlockSpec((1, 2), lambda i: (0, 0))],
        out_specs=[v, v, v, v],
        out_shape=[sds, sds, sds, sds],
    )(sp0, sp1, sn0, sn1, degp8, degn8, ev0, ev1, acc0, acc1,
      base0, base1, c0, c1, coef)


@functools.lru_cache(maxsize=None)
def _make_deg_kernel(npad, nblk, ns):
    iters = -(-nblk // ns)
    rpt = npad // ns
    mesh = plsc.VectorSubcoreMesh(core_axis_name="c", subcore_axis_name="s")
    out_sds = jax.ShapeDtypeStruct((npad, HALF), jnp.float32)

    @functools.partial(
        pl.kernel,
        out_type=[out_sds, out_sds],
        mesh=mesh,
        scratch_types=[
            pltpu.VMEM((RB * LANES,), jnp.int32),
            pltpu.VMEM((RB * LANES, HALF), jnp.float32),
            pltpu.VMEM_SHARED((npad, HALF), jnp.float32),
            pltpu.SemaphoreType.DMA,
        ],
        compiler_params=pltpu.CompilerParams(use_tc_tiling_on_sc=False),
    )
    def deg_kernel(dst_hbm, ones_hbm, outp_hbm, outn_hbm,
                   idxd_v, ones_v, acc, sem):
        c = lax.axis_index("c")
        sid = lax.axis_index("s")

        def run(dst2, out):
            pltpu.sync_copy(ones_hbm.at[pl.ds(0, RB * LANES)], ones_v)
            pltpu.sync_copy(ones_hbm.at[pl.ds(sid * rpt, rpt)],
                            acc.at[pl.ds(sid * rpt, rpt)])
            plsc.subcore_barrier()

            def block_body(i, _):
                b = sid + i * ns

                @pl.when(b < nblk)
                def _():
                    pltpu.sync_copy(dst2.at[pl.ds(b * RB * LANES,
                                                  RB * LANES)], idxd_v)
                    pltpu.async_copy(ones_v, acc.at[idxd_v], sem,
                                     add=True).wait()
                return 0

            lax.fori_loop(0, iters, block_body, 0)
            plsc.subcore_barrier()
            pltpu.sync_copy(acc.at[pl.ds(sid * rpt, rpt)],
                            out.at[pl.ds(sid * rpt, rpt)])

        @pl.when(c == 0)
        def _():
            run(dst_hbm.at[0], outp_hbm)

        @pl.when(c == 1)
        def _():
            run(dst_hbm.at[1], outn_hbm)

    return deg_kernel


@functools.lru_cache(maxsize=None)
def _make_msg_kernel(npad, nblk, ns):
    iters = -(-nblk // ns)
    rpt = npad // ns
    mesh = plsc.VectorSubcoreMesh(core_axis_name="c", subcore_axis_name="s")
    out_sds = jax.ShapeDtypeStruct((npad, HALF), jnp.float32)

    @functools.partial(
        pl.kernel,
        out_type=[out_sds, out_sds, out_sds, out_sds],
        mesh=mesh,
        scratch_types=[
            pltpu.VMEM((RB, LANES), jnp.int32),
            pltpu.VMEM((RB, LANES), jnp.int32),
            pltpu.VMEM((RB * LANES, HALF), jnp.float32),
            pltpu.VMEM_SHARED((npad, HALF), jnp.float32),
            pltpu.SemaphoreType.DMA,
            pltpu.SemaphoreType.DMA,
        ],
        compiler_params=pltpu.CompilerParams(use_tc_tiling_on_sc=False),
    )
    def msg_kernel(tp0_hbm, tp1_hbm, tn0_hbm, tn1_hbm, src_hbm, dst_hbm,
                   sp0_hbm, sp1_hbm, sn0_hbm, sn1_hbm,
                   idxs_v, idxd_v, rows_v, acc, sem_g, sem_s):
        c = lax.axis_index("c")
        sid = lax.axis_index("s")

        def run_sign(tbl, out, s):
            pltpu.sync_copy(tbl.at[pl.ds(sid * rpt, rpt)],
                            acc.at[pl.ds(sid * rpt, rpt)])
            plsc.subcore_barrier()

            def block_body(i, _):
                b = sid + i * ns

                @pl.when(b < nblk)
                def _():
                    pltpu.sync_copy(src_hbm.at[s].at[b], idxs_v)
                    pltpu.sync_copy(dst_hbm.at[s].at[b], idxd_v)
                    gd = [
                        pltpu.async_copy(
                            tbl.at[idxs_v.at[j]],
                            rows_v.at[pl.ds(j * LANES, LANES)], sem_g)
                        for j in range(RB)
                    ]
                    sd = []
                    for j in range(RB):
                        gd[j].wait()
                        sd.append(pltpu.async_copy(
                            rows_v.at[pl.ds(j * LANES, LANES)],
                            acc.at[idxd_v.at[j]], sem_s, add=True))
                    for d in sd:
                        d.wait()
                return 0

            lax.fori_loop(0, iters, block_body, 0)
            plsc.subcore_barrier()
            pltpu.sync_copy(acc.at[pl.ds(sid * rpt, rpt)],
                            out.at[pl.ds(sid * rpt, rpt)])
            plsc.subcore_barrier()

        @pl.when(c == 0)
        def _():
            run_sign(tp0_hbm, sp0_hbm, 0)
            run_sign(tn0_hbm, sn0_hbm, 1)

        @pl.when(c == 1)
        def _():
            run_sign(tp1_hbm, sp1_hbm, 0)
            run_sign(tn1_hbm, sn1_hbm, 1)

    return msg_kernel


def _tile8(vec16):
    return jnp.tile(vec16, 8).reshape(1, LANES)


def kernel(x, edge_index_pos, edge_index_neg, t, W_enc, b_enc, fn_g, fn_b,
           ln_g, ln_b, W_pos, b_pos, W_neg, b_neg, W_psi):
    n, _ = x.shape
    h = W_enc.shape[1]
    e = edge_index_pos.shape[1]
    blk_e = RB * LANES
    nblk = -(-e // blk_e)
    epad = nblk * blk_e - e
    npad = -(-n // LANES) * LANES
    m = npad * HALF // LANES
    info = plsc.get_sparse_core_info()
    ns = info.num_subcores
    bm = min(m, 1088)
    while m % bm or bm % 8:
        bm -= 1

    src_all = jnp.stack([edge_index_pos[0], edge_index_neg[0]])
    dst_all = jnp.stack([edge_index_pos[1], edge_index_neg[1]])
    if epad:
        src_all = jnp.pad(src_all, ((0, 0), (0, epad)))
        dst_all = jnp.pad(dst_all, ((0, 0), (0, epad)),
                          constant_values=jnp.int32(n))
    src_4d = src_all.reshape(2, nblk, RB, LANES)
    dst_4d = dst_all.reshape(2, nblk, RB, LANES)
    dst_flat = dst_all.reshape(2, nblk * RB * LANES)
    ones_p = jnp.ones((m, LANES), jnp.float32).reshape(npad, HALF)

    degp, degn = _make_deg_kernel(npad, nblk, ns)(dst_flat, ones_p)
    degp8 = degp.reshape(m, LANES)
    degn8 = degn.reshape(m, LANES)

    ucat, cvec = _fold_weights(W_pos, W_neg, W_psi, b_pos, b_neg)
    upos, uneg = ucat[:, :h], ucat[:, h:]
    eye8 = jnp.eye(8, dtype=jnp.float32)
    amat = jnp.kron(eye8, jnp.full((HALF, HALF), 1.0 / h, jnp.float32))
    bd = jnp.stack([
        jnp.kron(eye8, upos[:HALF, :HALF]),
        jnp.kron(eye8, upos[HALF:, :HALF]),
        jnp.kron(eye8, upos[:HALF, HALF:]),
        jnp.kron(eye8, upos[HALF:, HALF:]),
        jnp.kron(eye8, uneg[:HALF, :HALF]),
        jnp.kron(eye8, uneg[HALF:, :HALF]),
        jnp.kron(eye8, uneg[:HALF, HALF:]),
        jnp.kron(eye8, uneg[HALF:, HALF:]),
    ])
    g0 = _tile8(ln_g[:HALF])
    g1 = _tile8(ln_g[HALF:])
    b0 = _tile8(ln_b[:HALF])
    b1 = _tile8(ln_b[HALF:])
    c0 = _tile8(cvec[0, :HALF])
    c1 = _tile8(cvec[0, HALF:])

    h0 = _encode(x, W_enc, b_enc, fn_g, fn_b, 2000)
    pad_rows = ((0, npad - n), (0, 0))
    ev0 = jnp.pad(h0[:, :HALF], pad_rows).reshape(m, LANES)
    ev1 = jnp.pad(h0[:, HALF:], pad_rows).reshape(m, LANES)

    msg = _make_msg_kernel(npad, nblk, ns)

    dt = (t[1] - t[0]) / ODE_STEPS
    wts = (dt / 6.0, dt / 3.0, dt / 3.0, dt / 6.0)
    ats = (dt / 2.0, dt / 2.0, dt, dt * 0.0)

    for _ in range(ODE_STEPS):
        base0, base1 = ev0, ev1
        acc0, acc1 = base0, base1
        for si in range(4):
            tp0, tp1, tn0, tn1 = _pre_stage(
                ev0, ev1, degp8, degn8, amat, bd, g0, g1, b0, b1, bm)
            sp0, sp1, sn0, sn1 = msg(
                tp0.reshape(npad, HALF), tp1.reshape(npad, HALF),
                tn0.reshape(npad, HALF), tn1.reshape(npad, HALF),
                src_4d, dst_4d)
            coef = jnp.stack([wts[si], ats[si]]).reshape(1, 2)
            acc0, acc1, ev0, ev1 = _post_stage(
                sp0.reshape(m, LANES), sp1.reshape(m, LANES),
                sn0.reshape(m, LANES), sn1.reshape(m, LANES),
                degp8, degn8, ev0, ev1, acc0, acc1, base0, base1,
                c0, c1, coef, bm)
        ev0, ev1 = acc0, acc1

    hv0 = ev0.reshape(npad, HALF)[:n]
    hv1 = ev1.reshape(npad, HALF)[:n]
    return jnp.concatenate([hv0, hv1], axis=1)
